# Initial kernel scaffold; baseline (speedup 1.0000x reference)
import jax
import jax.numpy as jnp
from jax.experimental import pallas as pl


def kernel(x, edge_index, W1, b1, Wmu, bmu, Wls, bls, decW, decb, dec_mask):
    raise NotImplementedError("write your pallas kernel here")


# speedup vs baseline: 17.1563x; 17.1563x over previous
import functools

import jax
import jax.numpy as jnp
from jax import lax
from jax.experimental import pallas as pl
from jax.experimental.pallas import tpu as pltpu
from jax.experimental.pallas import tpu_sc as plsc

N = 10000
D = 128
NPAD = 10240
NC = 2
NS = 16
NW = NC * NS
E = 320000
CH = 128
EPW = NPAD
NCHUNK = EPW // CH
EPAD = NW * EPW
RPS = N // NS
ZR = NPAD // NS
DEGW = 128
BLK = 1000
GRID = N // BLK

_mesh = plsc.VectorSubcoreMesh(
    core_axis_name="c", subcore_axis_name="s", num_cores=NC, num_subcores=NS)


@functools.partial(
    pl.kernel,
    out_type=jax.ShapeDtypeStruct((NC, NPAD, DEGW), jnp.float32),
    mesh=_mesh,
    scratch_types=[
        pltpu.VMEM((NCHUNK, CH), jnp.int32),
        pltpu.VMEM((CH, DEGW), jnp.float32),
        pltpu.VMEM_SHARED((NPAD, DEGW), jnp.float32),
    ],
)
def _sc_deg(dst3_hbm, ones_hbm, zdeg_hbm, out_hbm, dstv, ones, acc):
    c = lax.axis_index("c")
    s = lax.axis_index("s")
    wid = c * NS + s
    pltpu.sync_copy(ones_hbm, ones)
    pltpu.sync_copy(dst3_hbm.at[wid], dstv)
    pltpu.sync_copy(zdeg_hbm, acc.at[pl.ds(s * ZR, ZR)])
    plsc.subcore_barrier()

    def body(j, _):
        pltpu.sync_copy(ones, acc.at[dstv.at[j]], add=True)
        return 0

    lax.fori_loop(0, NCHUNK, body, 0)
    plsc.subcore_barrier()
    pltpu.sync_copy(acc.at[pl.ds(s * ZR, ZR)], out_hbm.at[c, pl.ds(s * ZR, ZR)])


@functools.partial(
    pl.kernel,
    out_type=jax.ShapeDtypeStruct((NC, NPAD, D), jnp.float32),
    mesh=_mesh,
    scratch_types=[
        pltpu.VMEM((NCHUNK, CH), jnp.int32),
        pltpu.VMEM((NCHUNK, CH), jnp.int32),
        pltpu.VMEM((CH, D), jnp.float32),
        pltpu.VMEM_SHARED((NPAD, D), jnp.float32),
        pltpu.SemaphoreType.DMA,
    ],
)
def _sc_agg(src3_hbm, dst3_hbm, rows_hbm, zrow_hbm, out_hbm,
            srcv, dstv, buf0, acc, sem0):
    c = lax.axis_index("c")
    s = lax.axis_index("s")
    wid = c * NS + s
    pltpu.sync_copy(src3_hbm.at[wid], srcv)
    pltpu.sync_copy(dst3_hbm.at[wid], dstv)
    pltpu.sync_copy(zrow_hbm, acc.at[pl.ds(s * ZR, ZR)])
    plsc.subcore_barrier()

    def body(j, _):
        pltpu.async_copy(rows_hbm.at[srcv.at[j]], buf0, sem0).wait()
        pltpu.sync_copy(buf0, acc.at[dstv.at[j]], add=True)
        return 0

    lax.fori_loop(0, NCHUNK, body, 0)
    plsc.subcore_barrier()
    pltpu.sync_copy(acc.at[pl.ds(s * ZR, ZR)], out_hbm.at[c, pl.ds(s * ZR, ZR)])


def _prep_body(x_ref, w1_ref, degp_ref, xs_ref, dinv_ref):
    p = jnp.dot(x_ref[...], w1_ref[...], preferred_element_type=jnp.float32)
    deg = degp_ref[0, :, 0] + degp_ref[1, :, 0] + 1.0
    dinv = lax.rsqrt(deg)
    xs_ref[...] = p * dinv[:, None]
    dinv_ref[...] = jnp.broadcast_to(dinv[:, None], dinv_ref.shape)


def _mid_body(aggp_ref, xs_ref, dinv_ref, b1_ref, hs_ref):
    agg = aggp_ref[0] + aggp_ref[1]
    dinv = dinv_ref[:, 0:1]
    h = jnp.maximum(dinv * (agg + xs_ref[...]) + b1_ref[...], 0.0)
    hs_ref[...] = h * dinv


def _final_body(aggp_ref, hs_ref, dinv_ref, wmu_ref, bmu_ref, wls_ref, bls_ref,
                decw_ref, decb_ref, mask_ref, mu_ref, ls_ref, expr_ref):
    agg = aggp_ref[0] + aggp_ref[1]
    dinv = dinv_ref[:, 0:1]
    g = dinv * (agg + hs_ref[...])
    mu = jnp.dot(g, wmu_ref[...], preferred_element_type=jnp.float32) + bmu_ref[...]
    mu_ref[...] = mu
    ls_ref[...] = jnp.dot(g, wls_ref[...], preferred_element_type=jnp.float32) + bls_ref[...]
    t = jnp.dot(mu, decw_ref[...] * mask_ref[...],
                preferred_element_type=jnp.float32) + decb_ref[...]
    t = t - jnp.max(t, axis=-1, keepdims=True)
    e = jnp.exp(t)
    expr_ref[...] = e / jnp.sum(e, axis=-1, keepdims=True)


def _adj_body(a_ref, b_ref, out_ref):
    out_ref[...] = lax.dot_general(
        a_ref[...], b_ref[...], (((1,), (1,)), ((), ())),
        preferred_element_type=jnp.float32)


_prep = pl.pallas_call(
    _prep_body,
    grid=(GRID,),
    in_specs=[
        pl.BlockSpec((BLK, D), lambda i: (i, 0)),
        pl.BlockSpec((D, D), lambda i: (0, 0)),
        pl.BlockSpec((NC, BLK, DEGW), lambda i: (0, i, 0)),
    ],
    out_specs=[
        pl.BlockSpec((BLK, D), lambda i: (i, 0)),
        pl.BlockSpec((BLK, 8), lambda i: (i, 0)),
    ],
    out_shape=[
        jax.ShapeDtypeStruct((N, D), jnp.float32),
        jax.ShapeDtypeStruct((N, 8), jnp.float32),
    ],
)

_mid = pl.pallas_call(
    _mid_body,
    grid=(GRID,),
    in_specs=[
        pl.BlockSpec((NC, BLK, D), lambda i: (0, i, 0)),
        pl.BlockSpec((BLK, D), lambda i: (i, 0)),
        pl.BlockSpec((BLK, 8), lambda i: (i, 0)),
        pl.BlockSpec((1, D), lambda i: (0, 0)),
    ],
    out_specs=pl.BlockSpec((BLK, D), lambda i: (i, 0)),
    out_shape=jax.ShapeDtypeStruct((N, D), jnp.float32),
)

_final = pl.pallas_call(
    _final_body,
    grid=(GRID,),
    in_specs=[
        pl.BlockSpec((NC, BLK, D), lambda i: (0, i, 0)),
        pl.BlockSpec((BLK, D), lambda i: (i, 0)),
        pl.BlockSpec((BLK, 8), lambda i: (i, 0)),
        pl.BlockSpec((D, 32), lambda i: (0, 0)),
        pl.BlockSpec((1, 32), lambda i: (0, 0)),
        pl.BlockSpec((D, 32), lambda i: (0, 0)),
        pl.BlockSpec((1, 32), lambda i: (0, 0)),
        pl.BlockSpec((32, D), lambda i: (0, 0)),
        pl.BlockSpec((1, D), lambda i: (0, 0)),
        pl.BlockSpec((32, D), lambda i: (0, 0)),
    ],
    out_specs=[
        pl.BlockSpec((BLK, 32), lambda i: (i, 0)),
        pl.BlockSpec((BLK, 32), lambda i: (i, 0)),
        pl.BlockSpec((BLK, D), lambda i: (i, 0)),
    ],
    out_shape=[
        jax.ShapeDtypeStruct((N, 32), jnp.float32),
        jax.ShapeDtypeStruct((N, 32), jnp.float32),
        jax.ShapeDtypeStruct((N, D), jnp.float32),
    ],
)

ABLK = 1024
AGRID = (N + ABLK - 1) // ABLK

_adj = pl.pallas_call(
    _adj_body,
    grid=(AGRID, AGRID),
    in_specs=[
        pl.BlockSpec((ABLK, 32), lambda i, j: (i, 0)),
        pl.BlockSpec((ABLK, 32), lambda i, j: (j, 0)),
    ],
    out_specs=pl.BlockSpec((ABLK, ABLK), lambda i, j: (i, j)),
    out_shape=jax.ShapeDtypeStruct((N, N), jnp.float32),
)


def kernel(x, edge_index, W1, b1, Wmu, bmu, Wls, bls, decW, decb, dec_mask):
    src = edge_index[0]
    dst = edge_index[1]
    pad = EPAD - E
    pad_i = jnp.arange(pad, dtype=jnp.int32)
    src3 = jnp.concatenate([src, pad_i % N]).reshape(NW, NCHUNK, CH)
    dst3 = jnp.concatenate([dst, N + pad_i % (NPAD - N)]).reshape(NW, NCHUNK, CH)
    zdeg = jnp.zeros((ZR, DEGW), jnp.float32)
    zrow = jnp.zeros((ZR, D), jnp.float32)

    ones = jnp.ones((CH, DEGW), jnp.float32)
    degp = _sc_deg(dst3, ones, zdeg)
    xs, dinv = _prep(x, W1, degp)
    agg1p = _sc_agg(src3, dst3, xs, zrow)
    hs = _mid(agg1p, xs, dinv, b1.reshape(1, D))
    agg2p = _sc_agg(src3, dst3, hs, zrow)
    mu, logstd, expr = _final(
        agg2p, hs, dinv, Wmu, bmu.reshape(1, 32), Wls, bls.reshape(1, 32),
        decW, decb.reshape(1, D), dec_mask)
    adj = _adj(mu, mu)
    return (adj, expr, mu, logstd)


# speedup vs baseline: 18.4790x; 1.0771x over previous
import functools

import jax
import jax.numpy as jnp
from jax import lax
from jax.experimental import pallas as pl
from jax.experimental.pallas import tpu as pltpu
from jax.experimental.pallas import tpu_sc as plsc

N = 10000
D = 128
NPAD = 10240
NC = 2
NS = 16
NW = NC * NS
E = 320000
CH = 128
EPW = NPAD
NCHUNK = EPW // CH
EPAD = NW * EPW
RPS = N // NS
ZR = NPAD // NS
DEGW = 128
BLK = 1000
GRID = N // BLK

_mesh = plsc.VectorSubcoreMesh(
    core_axis_name="c", subcore_axis_name="s", num_cores=NC, num_subcores=NS)


@functools.partial(
    pl.kernel,
    out_type=jax.ShapeDtypeStruct((NC, NPAD, DEGW), jnp.float32),
    mesh=_mesh,
    scratch_types=[
        pltpu.VMEM((NCHUNK, CH), jnp.int32),
        pltpu.VMEM((CH, DEGW), jnp.float32),
        pltpu.VMEM_SHARED((NPAD, DEGW), jnp.float32),
    ],
)
def _sc_deg(dst3_hbm, ones_hbm, zdeg_hbm, out_hbm, dstv, ones, acc):
    c = lax.axis_index("c")
    s = lax.axis_index("s")
    wid = c * NS + s
    pltpu.sync_copy(ones_hbm, ones)
    pltpu.sync_copy(dst3_hbm.at[wid], dstv)
    pltpu.sync_copy(zdeg_hbm, acc.at[pl.ds(s * ZR, ZR)])
    plsc.subcore_barrier()

    def body(j, _):
        pltpu.sync_copy(ones, acc.at[dstv.at[j]], add=True)
        return 0

    lax.fori_loop(0, NCHUNK, body, 0)
    plsc.subcore_barrier()
    pltpu.sync_copy(acc.at[pl.ds(s * ZR, ZR)], out_hbm.at[c, pl.ds(s * ZR, ZR)])


@functools.partial(
    pl.kernel,
    out_type=jax.ShapeDtypeStruct((NC, NPAD, D), jnp.float32),
    mesh=_mesh,
    scratch_types=[
        pltpu.VMEM((NCHUNK // 2, CH), jnp.int32),
        pltpu.VMEM((NCHUNK // 2, CH), jnp.int32),
        pltpu.VMEM((CH, D), jnp.float32),
        pltpu.VMEM((CH, D), jnp.float32),
        pltpu.VMEM_SHARED((NPAD, D), jnp.float32),
        pltpu.SemaphoreType.DMA,
        pltpu.SemaphoreType.DMA,
    ],
)
def _sc_agg(src3_hbm, dst3_hbm, rows_hbm, zrow_hbm, out_hbm,
            srcv, dstv, buf0, buf1, acc, sem0, sem1):
    c = lax.axis_index("c")
    s = lax.axis_index("s")
    wid = c * NS + s
    pltpu.sync_copy(zrow_hbm, acc.at[pl.ds(s * ZR, ZR)])
    plsc.subcore_barrier()

    for half in range(2):
        hbase = half * (NCHUNK // 2)
        pltpu.sync_copy(src3_hbm.at[wid, pl.ds(hbase, NCHUNK // 2)], srcv)
        pltpu.sync_copy(dst3_hbm.at[wid, pl.ds(hbase, NCHUNK // 2)], dstv)

        def body(t, _):
            j0 = 2 * t
            pltpu.async_copy(rows_hbm.at[srcv.at[j0]], buf0, sem0)
            pltpu.async_copy(rows_hbm.at[srcv.at[j0 + 1]], buf1, sem1)
            pltpu.make_async_copy(rows_hbm.at[srcv.at[j0]], buf0, sem0).wait()
            pltpu.sync_copy(buf0, acc.at[dstv.at[j0]], add=True)
            pltpu.make_async_copy(rows_hbm.at[srcv.at[j0 + 1]], buf1, sem1).wait()
            pltpu.sync_copy(buf1, acc.at[dstv.at[j0 + 1]], add=True)
            return 0

        lax.fori_loop(0, NCHUNK // 4, body, 0)
    plsc.subcore_barrier()
    pltpu.sync_copy(acc.at[pl.ds(s * ZR, ZR)], out_hbm.at[c, pl.ds(s * ZR, ZR)])


def _prep_body(x_ref, w1_ref, degp_ref, xs_ref, dinv_ref):
    p = jnp.dot(x_ref[...], w1_ref[...], preferred_element_type=jnp.float32)
    deg = degp_ref[0, :, 0] + degp_ref[1, :, 0] + 1.0
    dinv = lax.rsqrt(deg)
    xs_ref[...] = p * dinv[:, None]
    dinv_ref[...] = jnp.broadcast_to(dinv[:, None], dinv_ref.shape)


def _mid_body(aggp_ref, xs_ref, dinv_ref, b1_ref, hs_ref):
    agg = aggp_ref[0] + aggp_ref[1]
    dinv = dinv_ref[:, 0:1]
    h = jnp.maximum(dinv * (agg + xs_ref[...]) + b1_ref[...], 0.0)
    hs_ref[...] = h * dinv


def _final_body(aggp_ref, hs_ref, dinv_ref, wmu_ref, bmu_ref, wls_ref, bls_ref,
                decw_ref, decb_ref, mask_ref, mu_ref, ls_ref, expr_ref):
    agg = aggp_ref[0] + aggp_ref[1]
    dinv = dinv_ref[:, 0:1]
    g = dinv * (agg + hs_ref[...])
    mu = jnp.dot(g, wmu_ref[...], preferred_element_type=jnp.float32) + bmu_ref[...]
    mu_ref[...] = mu
    ls_ref[...] = jnp.dot(g, wls_ref[...], preferred_element_type=jnp.float32) + bls_ref[...]
    t = jnp.dot(mu, decw_ref[...] * mask_ref[...],
                preferred_element_type=jnp.float32) + decb_ref[...]
    t = t - jnp.max(t, axis=-1, keepdims=True)
    e = jnp.exp(t)
    expr_ref[...] = e / jnp.sum(e, axis=-1, keepdims=True)


def _adj_body(a_ref, b_ref, out_ref):
    out_ref[...] = lax.dot_general(
        a_ref[...], b_ref[...], (((1,), (1,)), ((), ())),
        preferred_element_type=jnp.float32)


_prep = pl.pallas_call(
    _prep_body,
    grid=(GRID,),
    in_specs=[
        pl.BlockSpec((BLK, D), lambda i: (i, 0)),
        pl.BlockSpec((D, D), lambda i: (0, 0)),
        pl.BlockSpec((NC, BLK, DEGW), lambda i: (0, i, 0)),
    ],
    out_specs=[
        pl.BlockSpec((BLK, D), lambda i: (i, 0)),
        pl.BlockSpec((BLK, 8), lambda i: (i, 0)),
    ],
    out_shape=[
        jax.ShapeDtypeStruct((N, D), jnp.float32),
        jax.ShapeDtypeStruct((N, 8), jnp.float32),
    ],
)

_mid = pl.pallas_call(
    _mid_body,
    grid=(GRID,),
    in_specs=[
        pl.BlockSpec((NC, BLK, D), lambda i: (0, i, 0)),
        pl.BlockSpec((BLK, D), lambda i: (i, 0)),
        pl.BlockSpec((BLK, 8), lambda i: (i, 0)),
        pl.BlockSpec((1, D), lambda i: (0, 0)),
    ],
    out_specs=pl.BlockSpec((BLK, D), lambda i: (i, 0)),
    out_shape=jax.ShapeDtypeStruct((N, D), jnp.float32),
)

_final = pl.pallas_call(
    _final_body,
    grid=(GRID,),
    in_specs=[
        pl.BlockSpec((NC, BLK, D), lambda i: (0, i, 0)),
        pl.BlockSpec((BLK, D), lambda i: (i, 0)),
        pl.BlockSpec((BLK, 8), lambda i: (i, 0)),
        pl.BlockSpec((D, 32), lambda i: (0, 0)),
        pl.BlockSpec((1, 32), lambda i: (0, 0)),
        pl.BlockSpec((D, 32), lambda i: (0, 0)),
        pl.BlockSpec((1, 32), lambda i: (0, 0)),
        pl.BlockSpec((32, D), lambda i: (0, 0)),
        pl.BlockSpec((1, D), lambda i: (0, 0)),
        pl.BlockSpec((32, D), lambda i: (0, 0)),
    ],
    out_specs=[
        pl.BlockSpec((BLK, 32), lambda i: (i, 0)),
        pl.BlockSpec((BLK, 32), lambda i: (i, 0)),
        pl.BlockSpec((BLK, D), lambda i: (i, 0)),
    ],
    out_shape=[
        jax.ShapeDtypeStruct((N, 32), jnp.float32),
        jax.ShapeDtypeStruct((N, 32), jnp.float32),
        jax.ShapeDtypeStruct((N, D), jnp.float32),
    ],
)

ABLK = 1024
AGRID = (N + ABLK - 1) // ABLK

_adj = pl.pallas_call(
    _adj_body,
    grid=(AGRID, AGRID),
    in_specs=[
        pl.BlockSpec((ABLK, 32), lambda i, j: (i, 0)),
        pl.BlockSpec((ABLK, 32), lambda i, j: (j, 0)),
    ],
    out_specs=pl.BlockSpec((ABLK, ABLK), lambda i, j: (i, j)),
    out_shape=jax.ShapeDtypeStruct((N, N), jnp.float32),
)


def kernel(x, edge_index, W1, b1, Wmu, bmu, Wls, bls, decW, decb, dec_mask):
    src = edge_index[0]
    dst = edge_index[1]
    pad = EPAD - E
    pad_i = jnp.arange(pad, dtype=jnp.int32)
    src3 = jnp.concatenate([src, pad_i % N]).reshape(NW, NCHUNK, CH)
    dst3 = jnp.concatenate([dst, N + pad_i % (NPAD - N)]).reshape(NW, NCHUNK, CH)
    zdeg = jnp.zeros((ZR, DEGW), jnp.float32)
    zrow = jnp.zeros((ZR, D), jnp.float32)

    ones = jnp.ones((CH, DEGW), jnp.float32)
    degp = _sc_deg(dst3, ones, zdeg)
    xs, dinv = _prep(x, W1, degp)
    agg1p = _sc_agg(src3, dst3, xs, zrow)
    hs = _mid(agg1p, xs, dinv, b1.reshape(1, D))
    agg2p = _sc_agg(src3, dst3, hs, zrow)
    mu, logstd, expr = _final(
        agg2p, hs, dinv, Wmu, bmu.reshape(1, 32), Wls, bls.reshape(1, 32),
        decW, decb.reshape(1, D), dec_mask)
    adj = _adj(mu, mu)
    return (adj, expr, mu, logstd)


# speedup vs baseline: 18.6476x; 1.0091x over previous
import functools

import jax
import jax.numpy as jnp
from jax import lax
from jax.experimental import pallas as pl
from jax.experimental.pallas import tpu as pltpu
from jax.experimental.pallas import tpu_sc as plsc

N = 10000
D = 128
NPAD = 10240
NC = 2
NS = 16
NW = NC * NS
E = 320000
CH = 128
EPW = NPAD
NCHUNK = EPW // CH
EPAD = NW * EPW
RPS = N // NS
ZR = NPAD // NS
DEGW = 128
BLK = 1000
GRID = N // BLK

_mesh = plsc.VectorSubcoreMesh(
    core_axis_name="c", subcore_axis_name="s", num_cores=NC, num_subcores=NS)


@functools.partial(
    pl.kernel,
    out_type=jax.ShapeDtypeStruct((NC, NPAD, DEGW), jnp.float32),
    mesh=_mesh,
    scratch_types=[
        pltpu.VMEM((NCHUNK, CH), jnp.int32),
        pltpu.VMEM((CH, DEGW), jnp.float32),
        pltpu.VMEM_SHARED((NPAD, DEGW), jnp.float32),
    ],
)
def _sc_deg(dst3_hbm, ones_hbm, zdeg_hbm, out_hbm, dstv, ones, acc):
    c = lax.axis_index("c")
    s = lax.axis_index("s")
    wid = c * NS + s
    pltpu.sync_copy(ones_hbm, ones)
    pltpu.sync_copy(dst3_hbm.at[wid], dstv)
    pltpu.sync_copy(zdeg_hbm, acc.at[pl.ds(s * ZR, ZR)])
    plsc.subcore_barrier()

    def body(j, _):
        pltpu.sync_copy(ones, acc.at[dstv.at[j]], add=True)
        return 0

    lax.fori_loop(0, NCHUNK, body, 0)
    plsc.subcore_barrier()
    pltpu.sync_copy(acc.at[pl.ds(s * ZR, ZR)], out_hbm.at[c, pl.ds(s * ZR, ZR)])


@functools.partial(
    pl.kernel,
    out_type=jax.ShapeDtypeStruct((NC, NPAD, D), jnp.float32),
    mesh=_mesh,
    scratch_types=[
        pltpu.VMEM((NCHUNK // 2, CH), jnp.int32),
        pltpu.VMEM((NCHUNK // 2, CH), jnp.int32),
        pltpu.VMEM((CH, D), jnp.float32),
        pltpu.VMEM((CH, D), jnp.float32),
        pltpu.VMEM_SHARED((NPAD, D), jnp.float32),
        pltpu.SemaphoreType.DMA,
        pltpu.SemaphoreType.DMA,
        pltpu.SemaphoreType.DMA,
        pltpu.SemaphoreType.DMA,
    ],
)
def _sc_agg(src3_hbm, dst3_hbm, rows_hbm, zrow_hbm, out_hbm,
            srcv, dstv, buf0, buf1, acc, sem0, sem1, sem2, sem3):
    c = lax.axis_index("c")
    s = lax.axis_index("s")
    wid = c * NS + s
    pltpu.sync_copy(zrow_hbm, acc.at[pl.ds(s * ZR, ZR)])
    plsc.subcore_barrier()

    for half in range(2):
        hbase = half * (NCHUNK // 2)
        pltpu.sync_copy(src3_hbm.at[wid, pl.ds(hbase, NCHUNK // 2)], srcv)
        pltpu.sync_copy(dst3_hbm.at[wid, pl.ds(hbase, NCHUNK // 2)], dstv)

        def body(t, _):
            j0 = 2 * t
            g0 = pltpu.async_copy(rows_hbm.at[srcv.at[j0]], buf0, sem0)
            g1 = pltpu.async_copy(rows_hbm.at[srcv.at[j0 + 1]], buf1, sem1)
            g0.wait()
            s0 = pltpu.async_copy(buf0, acc.at[dstv.at[j0]], sem2, add=True)
            g1.wait()
            s1 = pltpu.async_copy(buf1, acc.at[dstv.at[j0 + 1]], sem3, add=True)
            s0.wait()
            s1.wait()
            return 0

        lax.fori_loop(0, NCHUNK // 4, body, 0)
    plsc.subcore_barrier()
    pltpu.sync_copy(acc.at[pl.ds(s * ZR, ZR)], out_hbm.at[c, pl.ds(s * ZR, ZR)])


def _prep_body(x_ref, w1_ref, degp_ref, xs_ref, dinv_ref):
    p = jnp.dot(x_ref[...], w1_ref[...], preferred_element_type=jnp.float32)
    deg = degp_ref[0, :, 0] + degp_ref[1, :, 0] + 1.0
    dinv = lax.rsqrt(deg)
    xs_ref[...] = p * dinv[:, None]
    dinv_ref[...] = jnp.broadcast_to(dinv[:, None], dinv_ref.shape)


def _mid_body(aggp_ref, xs_ref, dinv_ref, b1_ref, hs_ref):
    agg = aggp_ref[0] + aggp_ref[1]
    dinv = dinv_ref[:, 0:1]
    h = jnp.maximum(dinv * (agg + xs_ref[...]) + b1_ref[...], 0.0)
    hs_ref[...] = h * dinv


def _final_body(aggp_ref, hs_ref, dinv_ref, wmu_ref, bmu_ref, wls_ref, bls_ref,
                decw_ref, decb_ref, mask_ref, mu_ref, ls_ref, expr_ref):
    agg = aggp_ref[0] + aggp_ref[1]
    dinv = dinv_ref[:, 0:1]
    g = dinv * (agg + hs_ref[...])
    mu = jnp.dot(g, wmu_ref[...], preferred_element_type=jnp.float32) + bmu_ref[...]
    mu_ref[...] = mu
    ls_ref[...] = jnp.dot(g, wls_ref[...], preferred_element_type=jnp.float32) + bls_ref[...]
    t = jnp.dot(mu, decw_ref[...] * mask_ref[...],
                preferred_element_type=jnp.float32) + decb_ref[...]
    t = t - jnp.max(t, axis=-1, keepdims=True)
    e = jnp.exp(t)
    expr_ref[...] = e / jnp.sum(e, axis=-1, keepdims=True)


def _adj_body(a_ref, b_ref, out_ref):
    out_ref[...] = lax.dot_general(
        a_ref[...], b_ref[...], (((1,), (1,)), ((), ())),
        preferred_element_type=jnp.float32)


_prep = pl.pallas_call(
    _prep_body,
    grid=(GRID,),
    in_specs=[
        pl.BlockSpec((BLK, D), lambda i: (i, 0)),
        pl.BlockSpec((D, D), lambda i: (0, 0)),
        pl.BlockSpec((NC, BLK, DEGW), lambda i: (0, i, 0)),
    ],
    out_specs=[
        pl.BlockSpec((BLK, D), lambda i: (i, 0)),
        pl.BlockSpec((BLK, 8), lambda i: (i, 0)),
    ],
    out_shape=[
        jax.ShapeDtypeStruct((N, D), jnp.float32),
        jax.ShapeDtypeStruct((N, 8), jnp.float32),
    ],
)

_mid = pl.pallas_call(
    _mid_body,
    grid=(GRID,),
    in_specs=[
        pl.BlockSpec((NC, BLK, D), lambda i: (0, i, 0)),
        pl.BlockSpec((BLK, D), lambda i: (i, 0)),
        pl.BlockSpec((BLK, 8), lambda i: (i, 0)),
        pl.BlockSpec((1, D), lambda i: (0, 0)),
    ],
    out_specs=pl.BlockSpec((BLK, D), lambda i: (i, 0)),
    out_shape=jax.ShapeDtypeStruct((N, D), jnp.float32),
)

_final = pl.pallas_call(
    _final_body,
    grid=(GRID,),
    in_specs=[
        pl.BlockSpec((NC, BLK, D), lambda i: (0, i, 0)),
        pl.BlockSpec((BLK, D), lambda i: (i, 0)),
        pl.BlockSpec((BLK, 8), lambda i: (i, 0)),
        pl.BlockSpec((D, 32), lambda i: (0, 0)),
        pl.BlockSpec((1, 32), lambda i: (0, 0)),
        pl.BlockSpec((D, 32), lambda i: (0, 0)),
        pl.BlockSpec((1, 32), lambda i: (0, 0)),
        pl.BlockSpec((32, D), lambda i: (0, 0)),
        pl.BlockSpec((1, D), lambda i: (0, 0)),
        pl.BlockSpec((32, D), lambda i: (0, 0)),
    ],
    out_specs=[
        pl.BlockSpec((BLK, 32), lambda i: (i, 0)),
        pl.BlockSpec((BLK, 32), lambda i: (i, 0)),
        pl.BlockSpec((BLK, D), lambda i: (i, 0)),
    ],
    out_shape=[
        jax.ShapeDtypeStruct((N, 32), jnp.float32),
        jax.ShapeDtypeStruct((N, 32), jnp.float32),
        jax.ShapeDtypeStruct((N, D), jnp.float32),
    ],
)

ABLK = 1024
AGRID = (N + ABLK - 1) // ABLK

_adj = pl.pallas_call(
    _adj_body,
    grid=(AGRID, AGRID),
    in_specs=[
        pl.BlockSpec((ABLK, 32), lambda i, j: (i, 0)),
        pl.BlockSpec((ABLK, 32), lambda i, j: (j, 0)),
    ],
    out_specs=pl.BlockSpec((ABLK, ABLK), lambda i, j: (i, j)),
    out_shape=jax.ShapeDtypeStruct((N, N), jnp.float32),
)


def kernel(x, edge_index, W1, b1, Wmu, bmu, Wls, bls, decW, decb, dec_mask):
    src = edge_index[0]
    dst = edge_index[1]
    pad = EPAD - E
    pad_i = jnp.arange(pad, dtype=jnp.int32)
    src3 = jnp.concatenate([src, pad_i % N]).reshape(NW, NCHUNK, CH)
    dst3 = jnp.concatenate([dst, N + pad_i % (NPAD - N)]).reshape(NW, NCHUNK, CH)
    zdeg = jnp.zeros((ZR, DEGW), jnp.float32)
    zrow = jnp.zeros((ZR, D), jnp.float32)

    ones = jnp.ones((CH, DEGW), jnp.float32)
    degp = _sc_deg(dst3, ones, zdeg)
    xs, dinv = _prep(x, W1, degp)
    agg1p = _sc_agg(src3, dst3, xs, zrow)
    hs = _mid(agg1p, xs, dinv, b1.reshape(1, D))
    agg2p = _sc_agg(src3, dst3, hs, zrow)
    mu, logstd, expr = _final(
        agg2p, hs, dinv, Wmu, bmu.reshape(1, 32), Wls, bls.reshape(1, 32),
        decW, decb.reshape(1, D), dec_mask)
    adj = _adj(mu, mu)
    return (adj, expr, mu, logstd)


# speedup vs baseline: 20.0487x; 1.0751x over previous
import functools

import jax
import jax.numpy as jnp
from jax import lax
from jax.experimental import pallas as pl
from jax.experimental.pallas import tpu as pltpu
from jax.experimental.pallas import tpu_sc as plsc

N = 10000
D = 128
NPAD = 10240
NC = 2
NS = 16
NW = NC * NS
E = 320000
CH = 128
EPW = NPAD
NCHUNK = EPW // CH
EPAD = NW * EPW
RPS = N // NS
ZR = NPAD // NS
DEGW = 128
BLK = 1000
GRID = N // BLK

_mesh = plsc.VectorSubcoreMesh(
    core_axis_name="c", subcore_axis_name="s", num_cores=NC, num_subcores=NS)


@functools.partial(
    pl.kernel,
    out_type=jax.ShapeDtypeStruct((NC, NPAD, DEGW), jnp.float32),
    mesh=_mesh,
    scratch_types=[
        pltpu.VMEM((NCHUNK, CH), jnp.int32),
        pltpu.VMEM((CH, DEGW), jnp.float32),
        pltpu.VMEM_SHARED((NPAD, DEGW), jnp.float32),
    ],
)
def _sc_deg(dst3_hbm, ones_hbm, zdeg_hbm, out_hbm, dstv, ones, acc):
    c = lax.axis_index("c")
    s = lax.axis_index("s")
    wid = c * NS + s
    pltpu.sync_copy(ones_hbm, ones)
    pltpu.sync_copy(dst3_hbm.at[wid], dstv)
    pltpu.sync_copy(zdeg_hbm, acc.at[pl.ds(s * ZR, ZR)])
    plsc.subcore_barrier()

    def body(j, _):
        pltpu.sync_copy(ones, acc.at[dstv.at[j]], add=True)
        return 0

    lax.fori_loop(0, NCHUNK, body, 0)
    plsc.subcore_barrier()
    pltpu.sync_copy(acc.at[pl.ds(s * ZR, ZR)], out_hbm.at[c, pl.ds(s * ZR, ZR)])


@functools.partial(
    pl.kernel,
    out_type=jax.ShapeDtypeStruct((NC, NPAD, D), jnp.float32),
    mesh=_mesh,
    scratch_types=[
        pltpu.VMEM((NCHUNK // 2, CH), jnp.int32),
        pltpu.VMEM((NCHUNK // 2, CH), jnp.int32),
        pltpu.VMEM((CH, D), jnp.float32),
        pltpu.VMEM((CH, D), jnp.float32),
        pltpu.VMEM_SHARED((NPAD, D), jnp.float32),
        pltpu.SemaphoreType.DMA,
        pltpu.SemaphoreType.DMA,
        pltpu.SemaphoreType.DMA,
        pltpu.SemaphoreType.DMA,
    ],
)
def _sc_agg(src3_hbm, dst3_hbm, rows_hbm, zrow_hbm, out_hbm,
            srcv, dstv, buf0, buf1, acc, sem0, sem1, sem2, sem3):
    c = lax.axis_index("c")
    s = lax.axis_index("s")
    wid = c * NS + s
    pltpu.sync_copy(zrow_hbm, acc.at[pl.ds(s * ZR, ZR)])
    plsc.subcore_barrier()

    for half in range(2):
        hbase = half * (NCHUNK // 2)
        pltpu.sync_copy(src3_hbm.at[wid, pl.ds(hbase, NCHUNK // 2)], srcv)
        pltpu.sync_copy(dst3_hbm.at[wid, pl.ds(hbase, NCHUNK // 2)], dstv)

        def body(t, _):
            j0 = 4 * t
            g0 = pltpu.async_copy(rows_hbm.at[srcv.at[j0]], buf0, sem0)
            g1 = pltpu.async_copy(rows_hbm.at[srcv.at[j0 + 1]], buf1, sem1)
            g0.wait()
            s0 = pltpu.async_copy(buf0, acc.at[dstv.at[j0]], sem2, add=True)
            g1.wait()
            s1 = pltpu.async_copy(buf1, acc.at[dstv.at[j0 + 1]], sem3, add=True)
            s0.wait()
            g2 = pltpu.async_copy(rows_hbm.at[srcv.at[j0 + 2]], buf0, sem0)
            s1.wait()
            g3 = pltpu.async_copy(rows_hbm.at[srcv.at[j0 + 3]], buf1, sem1)
            g2.wait()
            s2 = pltpu.async_copy(buf0, acc.at[dstv.at[j0 + 2]], sem2, add=True)
            g3.wait()
            s3 = pltpu.async_copy(buf1, acc.at[dstv.at[j0 + 3]], sem3, add=True)
            s2.wait()
            s3.wait()
            return 0

        lax.fori_loop(0, NCHUNK // 8, body, 0)
    plsc.subcore_barrier()
    pltpu.sync_copy(acc.at[pl.ds(s * ZR, ZR)], out_hbm.at[c, pl.ds(s * ZR, ZR)])


def _prep_body(x_ref, w1_ref, degp_ref, xs_ref, dinv_ref):
    p = jnp.dot(x_ref[...], w1_ref[...], preferred_element_type=jnp.float32)
    deg = degp_ref[0, :, 0] + degp_ref[1, :, 0] + 1.0
    dinv = lax.rsqrt(deg)
    xs_ref[...] = p * dinv[:, None]
    dinv_ref[...] = jnp.broadcast_to(dinv[:, None], dinv_ref.shape)


def _mid_body(aggp_ref, xs_ref, dinv_ref, b1_ref, hs_ref):
    agg = aggp_ref[0] + aggp_ref[1]
    dinv = dinv_ref[:, 0:1]
    h = jnp.maximum(dinv * (agg + xs_ref[...]) + b1_ref[...], 0.0)
    hs_ref[...] = h * dinv


def _final_body(aggp_ref, hs_ref, dinv_ref, wmu_ref, bmu_ref, wls_ref, bls_ref,
                decw_ref, decb_ref, mask_ref, mu_ref, ls_ref, expr_ref):
    agg = aggp_ref[0] + aggp_ref[1]
    dinv = dinv_ref[:, 0:1]
    g = dinv * (agg + hs_ref[...])
    mu = jnp.dot(g, wmu_ref[...], preferred_element_type=jnp.float32) + bmu_ref[...]
    mu_ref[...] = mu
    ls_ref[...] = jnp.dot(g, wls_ref[...], preferred_element_type=jnp.float32) + bls_ref[...]
    t = jnp.dot(mu, decw_ref[...] * mask_ref[...],
                preferred_element_type=jnp.float32) + decb_ref[...]
    t = t - jnp.max(t, axis=-1, keepdims=True)
    e = jnp.exp(t)
    expr_ref[...] = e / jnp.sum(e, axis=-1, keepdims=True)


def _adj_body(a_ref, b_ref, out_ref):
    out_ref[...] = lax.dot_general(
        a_ref[...], b_ref[...], (((1,), (1,)), ((), ())),
        preferred_element_type=jnp.float32)


_prep = pl.pallas_call(
    _prep_body,
    grid=(GRID,),
    in_specs=[
        pl.BlockSpec((BLK, D), lambda i: (i, 0)),
        pl.BlockSpec((D, D), lambda i: (0, 0)),
        pl.BlockSpec((NC, BLK, DEGW), lambda i: (0, i, 0)),
    ],
    out_specs=[
        pl.BlockSpec((BLK, D), lambda i: (i, 0)),
        pl.BlockSpec((BLK, 8), lambda i: (i, 0)),
    ],
    out_shape=[
        jax.ShapeDtypeStruct((N, D), jnp.float32),
        jax.ShapeDtypeStruct((N, 8), jnp.float32),
    ],
)

_mid = pl.pallas_call(
    _mid_body,
    grid=(GRID,),
    in_specs=[
        pl.BlockSpec((NC, BLK, D), lambda i: (0, i, 0)),
        pl.BlockSpec((BLK, D), lambda i: (i, 0)),
        pl.BlockSpec((BLK, 8), lambda i: (i, 0)),
        pl.BlockSpec((1, D), lambda i: (0, 0)),
    ],
    out_specs=pl.BlockSpec((BLK, D), lambda i: (i, 0)),
    out_shape=jax.ShapeDtypeStruct((N, D), jnp.float32),
)

_final = pl.pallas_call(
    _final_body,
    grid=(GRID,),
    in_specs=[
        pl.BlockSpec((NC, BLK, D), lambda i: (0, i, 0)),
        pl.BlockSpec((BLK, D), lambda i: (i, 0)),
        pl.BlockSpec((BLK, 8), lambda i: (i, 0)),
        pl.BlockSpec((D, 32), lambda i: (0, 0)),
        pl.BlockSpec((1, 32), lambda i: (0, 0)),
        pl.BlockSpec((D, 32), lambda i: (0, 0)),
        pl.BlockSpec((1, 32), lambda i: (0, 0)),
        pl.BlockSpec((32, D), lambda i: (0, 0)),
        pl.BlockSpec((1, D), lambda i: (0, 0)),
        pl.BlockSpec((32, D), lambda i: (0, 0)),
    ],
    out_specs=[
        pl.BlockSpec((BLK, 32), lambda i: (i, 0)),
        pl.BlockSpec((BLK, 32), lambda i: (i, 0)),
        pl.BlockSpec((BLK, D), lambda i: (i, 0)),
    ],
    out_shape=[
        jax.ShapeDtypeStruct((N, 32), jnp.float32),
        jax.ShapeDtypeStruct((N, 32), jnp.float32),
        jax.ShapeDtypeStruct((N, D), jnp.float32),
    ],
)

ABLK = 1024
AGRID = (N + ABLK - 1) // ABLK

_adj = pl.pallas_call(
    _adj_body,
    grid=(AGRID, AGRID),
    in_specs=[
        pl.BlockSpec((ABLK, 32), lambda i, j: (i, 0)),
        pl.BlockSpec((ABLK, 32), lambda i, j: (j, 0)),
    ],
    out_specs=pl.BlockSpec((ABLK, ABLK), lambda i, j: (i, j)),
    out_shape=jax.ShapeDtypeStruct((N, N), jnp.float32),
)


def kernel(x, edge_index, W1, b1, Wmu, bmu, Wls, bls, decW, decb, dec_mask):
    src = edge_index[0]
    dst = edge_index[1]
    pad = EPAD - E
    pad_i = jnp.arange(pad, dtype=jnp.int32)
    src3 = jnp.concatenate([src, pad_i % N]).reshape(NW, NCHUNK, CH)
    dst3 = jnp.concatenate([dst, N + pad_i % (NPAD - N)]).reshape(NW, NCHUNK, CH)
    zdeg = jnp.zeros((ZR, DEGW), jnp.float32)
    zrow = jnp.zeros((ZR, D), jnp.float32)

    ones = jnp.ones((CH, DEGW), jnp.float32)
    degp = _sc_deg(dst3, ones, zdeg)
    xs, dinv = _prep(x, W1, degp)
    agg1p = _sc_agg(src3, dst3, xs, zrow)
    hs = _mid(agg1p, xs, dinv, b1.reshape(1, D))
    agg2p = _sc_agg(src3, dst3, hs, zrow)
    mu, logstd, expr = _final(
        agg2p, hs, dinv, Wmu, bmu.reshape(1, 32), Wls, bls.reshape(1, 32),
        decW, decb.reshape(1, D), dec_mask)
    adj = _adj(mu, mu)
    return (adj, expr, mu, logstd)
